# baseline (speedup 1.0000x reference)
import functools

import jax
import jax.numpy as jnp
from jax import lax
from jax.experimental import pallas as pl
from jax.experimental.pallas import tpu as pltpu
from jax.experimental.pallas import tpu_sc as plsc

EMB_DIM = 64
SCALE = 8.0
LANES = 16


def _emb_call(n_tokens, per_w, chunk, num_cores):
    n_chunks = per_w // chunk
    mesh = plsc.VectorSubcoreMesh(core_axis_name="c", subcore_axis_name="s")

    @functools.partial(
        pl.kernel,
        mesh=mesh,
        out_type=jax.ShapeDtypeStruct((n_tokens, EMB_DIM), jnp.float32),
        compiler_params=pltpu.CompilerParams(use_tc_tiling_on_sc=False),
        scratch_types=[
            pltpu.VMEM((per_w,), jnp.int32),
            pltpu.VMEM((chunk, EMB_DIM), jnp.float32),
            pltpu.SemaphoreType.DMA,
        ],
    )
    def emb_k(idx_hbm, tab_hbm, out_hbm, idx_v, rows_v, sem):
        wid = lax.axis_index("s") * num_cores + lax.axis_index("c")
        base = wid * per_w
        pltpu.sync_copy(idx_hbm.at[pl.ds(base, per_w)], idx_v)

        def chunk_body(c, carry):
            pltpu.async_copy(
                tab_hbm.at[idx_v.at[pl.ds(c * chunk, chunk)]], rows_v, sem
            ).wait()

            def row_body(r, carry2):
                for j in range(EMB_DIM // LANES):
                    sl = pl.ds(j * LANES, LANES)
                    rows_v[r, sl] = rows_v[r, sl] * SCALE
                return carry2

            lax.fori_loop(0, chunk, row_body, 0)
            pltpu.sync_copy(rows_v, out_hbm.at[pl.ds(base + c * chunk, chunk)])
            return carry

        lax.fori_loop(0, n_chunks, chunk_body, 0)

    return emb_k


def kernel(x, table):
    b, l = x.shape
    n_tokens = b * l
    info = plsc.get_sparse_core_info()
    n_workers = info.num_cores * info.num_subcores
    per_w = n_tokens // n_workers
    chunk = 512
    emb_k = _emb_call(n_tokens, per_w, chunk, info.num_cores)
    out = emb_k(x.reshape(n_tokens), table)
    return out.reshape(b, l, EMB_DIM)


# speedup vs baseline: 1.1130x; 1.1130x over previous
import functools

import jax
import jax.numpy as jnp
from jax import lax
from jax.experimental import pallas as pl
from jax.experimental.pallas import tpu as pltpu
from jax.experimental.pallas import tpu_sc as plsc

EMB_DIM = 64
SCALE = 8.0
LANES = 16
CHUNK = 256


def _emb_call(n_tokens, per_w, num_cores):
    n_chunks = per_w // CHUNK
    n_pairs = n_chunks // 2
    mesh = plsc.VectorSubcoreMesh(core_axis_name="c", subcore_axis_name="s")

    @functools.partial(
        pl.kernel,
        mesh=mesh,
        out_type=jax.ShapeDtypeStruct((n_tokens, EMB_DIM), jnp.float32),
        compiler_params=pltpu.CompilerParams(use_tc_tiling_on_sc=False),
        scratch_types=[
            pltpu.VMEM((per_w,), jnp.int32),
            pltpu.VMEM((CHUNK, EMB_DIM), jnp.float32),
            pltpu.VMEM((CHUNK, EMB_DIM), jnp.float32),
            pltpu.VMEM((CHUNK, EMB_DIM), jnp.float32),
            pltpu.VMEM((CHUNK, EMB_DIM), jnp.float32),
            pltpu.SemaphoreType.DMA,
            pltpu.SemaphoreType.DMA,
            pltpu.SemaphoreType.DMA,
            pltpu.SemaphoreType.DMA,
        ],
    )
    def emb_k(idx_hbm, tab_hbm, out_hbm, idx_v, in0, in1, ob0, ob1,
              gs0, gs1, ws0, ws1):
        wid = lax.axis_index("s") * num_cores + lax.axis_index("c")
        base = wid * per_w
        pltpu.sync_copy(idx_hbm.at[pl.ds(base, per_w)], idx_v)

        ins, outs = (in0, in1), (ob0, ob1)
        gsems, wsems = (gs0, gs1), (ws0, ws1)

        def gather(c, b):
            pltpu.async_copy(
                tab_hbm.at[idx_v.at[pl.ds(c * CHUNK, CHUNK)]], ins[b], gsems[b]
            )

        def gather_wait(b):
            pltpu.make_async_copy(
                tab_hbm.at[idx_v.at[pl.ds(0, CHUNK)]], ins[b], gsems[b]
            ).wait()

        def write(c, b):
            pltpu.async_copy(
                outs[b], out_hbm.at[pl.ds(base + c * CHUNK, CHUNK)], wsems[b]
            )

        def write_wait(b):
            pltpu.make_async_copy(
                outs[b], out_hbm.at[pl.ds(0, CHUNK)], wsems[b]
            ).wait()

        for b in range(2):
            gather(b, b)

        def pair_body(g, carry):
            for b in range(2):
                c = g * 2 + b
                gather_wait(b)

                @pl.when(g >= 1)
                def _():
                    write_wait(b)

                src, dst = ins[b], outs[b]

                def row_body(r, carry2):
                    for j in range(EMB_DIM // LANES):
                        sl = pl.ds(j * LANES, LANES)
                        dst[r, sl] = src[r, sl] * SCALE
                    return carry2

                lax.fori_loop(0, CHUNK, row_body, 0)

                write(c, b)

                @pl.when(g < n_pairs - 1)
                def _():
                    gather(c + 2, b)

            return carry

        lax.fori_loop(0, n_pairs, pair_body, 0)
        for b in range(2):
            write_wait(b)

    return emb_k


def kernel(x, table):
    b, l = x.shape
    n_tokens = b * l
    info = plsc.get_sparse_core_info()
    n_workers = info.num_cores * info.num_subcores
    per_w = n_tokens // n_workers
    emb_k = _emb_call(n_tokens, per_w, info.num_cores)
    out = emb_k(x.reshape(n_tokens), table)
    return out.reshape(b, l, EMB_DIM)


# speedup vs baseline: 1.1167x; 1.0034x over previous
import functools

import jax
import jax.numpy as jnp
from jax import lax
from jax.experimental import pallas as pl
from jax.experimental.pallas import tpu as pltpu
from jax.experimental.pallas import tpu_sc as plsc

EMB_DIM = 64
SCALE = 8.0
LANES = 16
CHUNK = 256


def _emb_call(n_tokens, per_w, num_cores):
    n_chunks = per_w // CHUNK
    n_pairs = n_chunks // 2
    mesh = plsc.VectorSubcoreMesh(core_axis_name="c", subcore_axis_name="s")

    @functools.partial(
        pl.kernel,
        mesh=mesh,
        out_type=jax.ShapeDtypeStruct((n_tokens, EMB_DIM), jnp.float32),
        compiler_params=pltpu.CompilerParams(use_tc_tiling_on_sc=False),
        scratch_types=[
            pltpu.VMEM((per_w,), jnp.int32),
            pltpu.VMEM((CHUNK, EMB_DIM), jnp.float32),
            pltpu.VMEM((CHUNK, EMB_DIM), jnp.float32),
            pltpu.VMEM((CHUNK, EMB_DIM), jnp.float32),
            pltpu.VMEM((CHUNK, EMB_DIM), jnp.float32),
            pltpu.SemaphoreType.DMA,
            pltpu.SemaphoreType.DMA,
            pltpu.SemaphoreType.DMA,
            pltpu.SemaphoreType.DMA,
        ],
    )
    def emb_k(idx_hbm, tab_hbm, out_hbm, idx_v, in0, in1, ob0, ob1,
              gs0, gs1, ws0, ws1):
        wid = lax.axis_index("s") * num_cores + lax.axis_index("c")
        base = wid * per_w
        pltpu.sync_copy(idx_hbm.at[pl.ds(base, per_w)], idx_v)

        ins, outs = (in0, in1), (ob0, ob1)
        gsems, wsems = (gs0, gs1), (ws0, ws1)

        def gather(c, b):
            pltpu.async_copy(
                tab_hbm.at[idx_v.at[pl.ds(c * CHUNK, CHUNK)]], ins[b], gsems[b]
            )

        def gather_wait(b):
            pltpu.make_async_copy(
                tab_hbm.at[idx_v.at[pl.ds(0, CHUNK)]], ins[b], gsems[b]
            ).wait()

        def write(c, b):
            pltpu.async_copy(
                outs[b], out_hbm.at[pl.ds(base + c * CHUNK, CHUNK)], wsems[b]
            )

        def write_wait(b):
            pltpu.make_async_copy(
                outs[b], out_hbm.at[pl.ds(0, CHUNK)], wsems[b]
            ).wait()

        for b in range(2):
            gather(b, b)

        def pair_body(g, carry):
            for b in range(2):
                c = g * 2 + b
                gather_wait(b)

                @pl.when(g >= 1)
                def _():
                    write_wait(b)

                src, dst = ins[b], outs[b]

                def row_body(r8, carry2):
                    r0 = r8 * 8
                    for k in range(8):
                        for j in range(EMB_DIM // LANES):
                            sl = pl.ds(j * LANES, LANES)
                            dst[r0 + k, sl] = src[r0 + k, sl] * SCALE
                    return carry2

                lax.fori_loop(0, CHUNK // 8, row_body, 0)

                write(c, b)

                @pl.when(g < n_pairs - 1)
                def _():
                    gather(c + 2, b)

            return carry

        lax.fori_loop(0, n_pairs, pair_body, 0)
        for b in range(2):
            write_wait(b)

    return emb_k


def kernel(x, table):
    b, l = x.shape
    n_tokens = b * l
    info = plsc.get_sparse_core_info()
    n_workers = info.num_cores * info.num_subcores
    per_w = n_tokens // n_workers
    emb_k = _emb_call(n_tokens, per_w, info.num_cores)
    out = emb_k(x.reshape(n_tokens), table)
    return out.reshape(b, l, EMB_DIM)
